# Initial kernel scaffold; baseline (speedup 1.0000x reference)
import jax
import jax.numpy as jnp
from jax.experimental import pallas as pl


def kernel(x, edge_index, W1, a_src1, a_dst1, b1, g1, be1, W2, a_src2, a_dst2, b2, g2, be2, W3, a_src3, a_dst3, b3, g3, be3):
    raise NotImplementedError("write your pallas kernel here")


# speedup vs baseline: 5.1666x; 5.1666x over previous
import dataclasses
import functools

import jax
import jax.numpy as jnp
from jax import lax
from jax.experimental import pallas as pl
from jax.experimental.pallas import tpu as pltpu
from jax.experimental.pallas import tpu_sc as plsc

N = 10000
HEADS = 4
HID = 256
MP = 10240
NC, NS = 2, 16
NW = NC * NS
LPW = 128


def _sc_params():
    cp = pltpu.CompilerParams()
    if "needs_layout_passes" in pltpu.CompilerParams.__dataclass_fields__:
        cp = dataclasses.replace(cp, needs_layout_passes=False)
    return cp


_MESH = plsc.VectorSubcoreMesh(core_axis_name="c", subcore_axis_name="s")


def _mm_body(x_ref, w_ref, asr_ref, adr_ref, h_ref, an_ref, ad_ref, *, H, C):
    h = lax.dot_general(
        x_ref[...], w_ref[...], (((1,), (0,)), ((), ())),
        precision=lax.Precision.HIGHEST,
        preferred_element_type=jnp.float32)
    h_ref[...] = h
    hh = h.reshape(h.shape[0], H, C)
    an_ref[...] = jnp.sum(hh * asr_ref[...][None], axis=-1)
    ad_ref[...] = jnp.sum(hh * adr_ref[...][None], axis=-1)


def _mm(z, W, asr, adr):
    K, F = W.shape
    H, C = asr.shape
    BM = 1024
    grid = (MP // BM,)
    return pl.pallas_call(
        functools.partial(_mm_body, H=H, C=C),
        grid=grid,
        in_specs=[
            pl.BlockSpec((BM, K), lambda i: (i, 0)),
            pl.BlockSpec((K, F), lambda i: (0, 0)),
            pl.BlockSpec((H, C), lambda i: (0, 0)),
            pl.BlockSpec((H, C), lambda i: (0, 0)),
        ],
        out_specs=[
            pl.BlockSpec((BM, F), lambda i: (i, 0)),
            pl.BlockSpec((BM, H), lambda i: (i, 0)),
            pl.BlockSpec((BM, H), lambda i: (i, 0)),
        ],
        out_shape=[
            jax.ShapeDtypeStruct((MP, F), jnp.float32),
            jax.ShapeDtypeStruct((MP, H), jnp.float32),
            jax.ShapeDtypeStruct((MP, H), jnp.float32),
        ],
    )(z, W, asr, adr)


def _sca_body(an_hbm, ad_hbm, src_hbm, dst_hbm, ep_hbm, sp_hbm,
              an_v, ad_v, src_v, dst_v, cb, idxb, zb, s_sh, *, H, G, BPT):
    cid = lax.axis_index("c")
    sid = lax.axis_index("s")
    wid = sid * NC + cid
    stripe = MP * H // NS

    pltpu.sync_copy(src_hbm.at[wid], src_v)
    pltpu.sync_copy(dst_hbm.at[wid], dst_v)

    @pl.loop(0, stripe // 16)
    def _(i):
        zb[pl.ds(i * 16, 16)] = jnp.zeros((16,), jnp.float32)

    pltpu.sync_copy(zb, s_sh.at[pl.ds(sid * stripe, stripe)])
    plsc.subcore_barrier()

    for p in range(H // G):
        pltpu.sync_copy(an_hbm.at[pl.ds(p * G * MP, G * MP)], an_v)
        pltpu.sync_copy(ad_hbm.at[pl.ds(p * G * MP, G * MP)], ad_v)

        @pl.loop(0, BPT)
        def _(b):
            for c in range(LPW // 16):
                sl = pl.ds(c * 16, 16)
                s16 = src_v[b, sl]
                d16 = dst_v[b, sl]
                for g in range(G):
                    ga = plsc.load_gather(an_v, [s16 + g * MP])
                    gd = plsc.load_gather(ad_v, [d16 + g * MP])
                    a = ga + gd
                    a = jnp.where(a > 0, a, a * 0.2)
                    cb[g, b, sl] = jnp.exp(a)

        for g in range(G):
            pltpu.sync_copy(cb.at[g], ep_hbm.at[p * G + g, wid])

        @pl.loop(0, BPT)
        def _(b):
            for g in range(G):
                for c in range(LPW // 16):
                    sl = pl.ds(c * 16, 16)
                    idxb[sl] = dst_v[b, sl] + (p * G + g) * MP
                pltpu.sync_copy(cb.at[g, b], s_sh.at[idxb], add=True)

    plsc.subcore_barrier()
    pltpu.sync_copy(s_sh.at[pl.ds(sid * stripe, stripe)],
                    sp_hbm.at[cid, pl.ds(sid * stripe, stripe)])


def _sca(anp, adp, src3, dst3, H):
    BPT = src3.shape[1]
    G = min(2, H)
    k = pl.kernel(
        functools.partial(_sca_body, H=H, G=G, BPT=BPT),
        out_type=[
            jax.ShapeDtypeStruct((H, NW, BPT, LPW), jnp.float32),
            jax.ShapeDtypeStruct((NC, MP * H), jnp.float32),
        ],
        mesh=_MESH,
        scratch_types=[
            pltpu.VMEM((G * MP,), jnp.float32),
            pltpu.VMEM((G * MP,), jnp.float32),
            pltpu.VMEM((BPT, LPW), jnp.int32),
            pltpu.VMEM((BPT, LPW), jnp.int32),
            pltpu.VMEM((G, BPT, LPW), jnp.float32),
            pltpu.VMEM((LPW,), jnp.int32),
            pltpu.VMEM((MP * H // NS,), jnp.float32),
            pltpu.VMEM_SHARED((MP * H,), jnp.float32),
        ],
        compiler_params=_sc_params(),
    )
    return k(anp, adp, src3, dst3)


def _scaa_body(ep_hbm, rcp_hbm, dst_hbm, ap_hbm,
               rcp_v, dst_v, e_v, a_v, *, H, BPT):
    cid = lax.axis_index("c")
    sid = lax.axis_index("s")
    wid = sid * NC + cid

    pltpu.sync_copy(rcp_hbm, rcp_v)
    pltpu.sync_copy(dst_hbm.at[wid], dst_v)
    for h in range(H):
        pltpu.sync_copy(ep_hbm.at[h, wid], e_v)

        @pl.loop(0, BPT)
        def _(b):
            for c in range(LPW // 16):
                sl = pl.ds(c * 16, 16)
                d16 = dst_v[b, sl]
                r16 = plsc.load_gather(rcp_v, [d16 + h * MP])
                a_v[b, sl] = e_v[b, sl] * r16

        pltpu.sync_copy(a_v, ap_hbm.at[h, wid])


def _scaa(ep, rcp_flat, dst3, H):
    BPT = dst3.shape[1]
    k = pl.kernel(
        functools.partial(_scaa_body, H=H, BPT=BPT),
        out_type=jax.ShapeDtypeStruct((H, NW, BPT, LPW), jnp.float32),
        mesh=_MESH,
        scratch_types=[
            pltpu.VMEM((MP * H,), jnp.float32),
            pltpu.VMEM((BPT, LPW), jnp.int32),
            pltpu.VMEM((BPT, LPW), jnp.float32),
            pltpu.VMEM((BPT, LPW), jnp.float32),
        ],
        compiler_params=_sc_params(),
    )
    return k(ep, rcp_flat, dst3)


def _recip_body(sp_ref, r_ref):
    r_ref[...] = 1.0 / (sp_ref[0] + sp_ref[1] + 1e-16)


def _recip(sp):
    sp2 = sp.reshape(NC, -1, 128)
    R = sp2.shape[1]
    out = pl.pallas_call(
        _recip_body,
        in_specs=[pl.BlockSpec((NC, R, 128), lambda: (0, 0, 0))],
        out_specs=pl.BlockSpec((R, 128), lambda: (0, 0)),
        out_shape=jax.ShapeDtypeStruct((R, 128), jnp.float32),
    )(sp2)
    return out.reshape(-1)


def _scb_body(h8_hbm, ap_hbm, src_hbm, dst_hbm, o_hbm,
              src_v, dst_v, a_v, rowb, zb, idxg, idxs, acc_sh, sem,
              *, S, H, BPT):
    cid = lax.axis_index("c")
    sid = lax.axis_index("s")
    wid = sid * NC + cid
    rstripe = MP // NS

    pltpu.sync_copy(src_hbm.at[wid], src_v)
    pltpu.sync_copy(dst_hbm.at[wid], dst_v)

    @pl.loop(0, 64)
    def _(i):
        for c in range(8):
            zb[i, pl.ds(c * 16, 16)] = jnp.zeros((16,), jnp.float32)

    for s in range(S):
        hh = s // (S // H)
        if s % (S // H) == 0:
            pltpu.sync_copy(ap_hbm.at[hh, wid], a_v)

        @pl.loop(0, rstripe // 64)
        def _(i):
            pltpu.sync_copy(zb, acc_sh.at[pl.ds(sid * rstripe + i * 64, 64)])

        plsc.subcore_barrier()

        @pl.loop(0, BPT * (LPW // 16))
        def _(b):
            row = b // (LPW // 16)
            sl = pl.ds((b % (LPW // 16)) * 16, 16)
            s16 = src_v[row, sl]
            d16 = dst_v[row, sl]
            idxg[...] = s16 * S + s
            idxs[...] = d16
            cp = pltpu.async_copy(h8_hbm.at[idxg], rowb, sem)
            rowv = jnp.zeros((16,), jnp.int32) + row
            colbase = (b % (LPW // 16)) * 16
            cp.wait()
            for i in range(16):
                colv = jnp.zeros((16,), jnp.int32) + (colbase + i)
                av = plsc.load_gather(a_v, [rowv, colv])
                for c in range(8):
                    csl = pl.ds(c * 16, 16)
                    rowb[i, csl] = rowb[i, csl] * av
            pltpu.sync_copy(rowb, acc_sh.at[idxs], add=True)

        plsc.subcore_barrier()
        pltpu.sync_copy(acc_sh.at[pl.ds(sid * rstripe, rstripe)],
                        o_hbm.at[cid, s, pl.ds(sid * rstripe, rstripe)])
        plsc.subcore_barrier()


def _scb(h8, ap, src3, dst3, S, H):
    BPT = src3.shape[1]
    k = pl.kernel(
        functools.partial(_scb_body, S=S, H=H, BPT=BPT),
        out_type=jax.ShapeDtypeStruct((NC, S, MP, 128), jnp.float32),
        mesh=_MESH,
        scratch_types=[
            pltpu.VMEM((BPT, LPW), jnp.int32),
            pltpu.VMEM((BPT, LPW), jnp.int32),
            pltpu.VMEM((BPT, LPW), jnp.float32),
            pltpu.VMEM((16, 128), jnp.float32),
            pltpu.VMEM((64, 128), jnp.float32),
            pltpu.VMEM((16,), jnp.int32),
            pltpu.VMEM((16,), jnp.int32),
            pltpu.VMEM_SHARED((MP, 128), jnp.float32),
            pltpu.SemaphoreType.DMA,
        ],
        compiler_params=_sc_params(),
    )
    return k(h8, ap, src3, dst3)


def _stats_body(o0_ref, o1_ref, b_ref, zp_ref, st_ref, *, RB):
    i = pl.program_id(1)
    z = o0_ref[0] + o1_ref[0] + b_ref[0]
    zp_ref[0] = z
    row = lax.broadcasted_iota(jnp.int32, (RB, 128), 0) + i * RB
    m = (row < N).astype(jnp.float32)
    zm = z * m

    @pl.when(i == 0)
    def _():
        st_ref[...] = jnp.zeros_like(st_ref)

    st_ref[0, 0:1, :] += jnp.sum(zm, axis=0, keepdims=True)
    st_ref[0, 1:2, :] += jnp.sum(zm * zm, axis=0, keepdims=True)


def _stats(o0, o1, b2):
    S = o0.shape[0]
    RB = 1024
    return pl.pallas_call(
        functools.partial(_stats_body, RB=RB),
        grid=(S, MP // RB),
        in_specs=[
            pl.BlockSpec((1, RB, 128), lambda s, i: (s, i, 0)),
            pl.BlockSpec((1, RB, 128), lambda s, i: (s, i, 0)),
            pl.BlockSpec((1, 1, 128), lambda s, i: (s, 0, 0)),
        ],
        out_specs=[
            pl.BlockSpec((1, RB, 128), lambda s, i: (s, i, 0)),
            pl.BlockSpec((1, 8, 128), lambda s, i: (s, 0, 0)),
        ],
        out_shape=[
            jax.ShapeDtypeStruct((S, MP, 128), jnp.float32),
            jax.ShapeDtypeStruct((S, 8, 128), jnp.float32),
        ],
    )(o0, o1, b2)


def _bn_body(zp_ref, st_ref, g_ref, be_ref, z_ref):
    mu = st_ref[0, 0:1, :] / N
    ex2 = st_ref[0, 1:2, :] / N
    rstd = lax.rsqrt(ex2 - mu * mu + 1e-5)
    z = (zp_ref[0] - mu) * rstd * g_ref[0] + be_ref[0]
    z_ref[...] = jnp.where(z > 0, z, 0.2 * z)


def _bn(zp, st, g2, be2):
    S = zp.shape[0]
    RB = 1024
    return pl.pallas_call(
        _bn_body,
        grid=(MP // RB, S),
        in_specs=[
            pl.BlockSpec((1, RB, 128), lambda i, s: (s, i, 0)),
            pl.BlockSpec((1, 8, 128), lambda i, s: (s, 0, 0)),
            pl.BlockSpec((1, 1, 128), lambda i, s: (s, 0, 0)),
            pl.BlockSpec((1, 1, 128), lambda i, s: (s, 0, 0)),
        ],
        out_specs=pl.BlockSpec((RB, 128), lambda i, s: (i, s)),
        out_shape=jax.ShapeDtypeStruct((MP, S * 128), jnp.float32),
    )(zp, st, g2, be2)


def _layer(z, src3, dst3, W, a_src, a_dst, b, g, be, H):
    K, F = W.shape
    C = F // H
    S = F // 128
    h, an, ad = _mm(z, W, a_src.reshape(H, C), a_dst.reshape(H, C))
    ep, sp = _sca(an.T.reshape(-1), ad.T.reshape(-1), src3, dst3, H)
    rcp = _recip(sp)
    ap = _scaa(ep, rcp, dst3, H)
    o = _scb(h.reshape(MP * S, 128), ap, src3, dst3, S, H)
    zp, st = _stats(o[0], o[1], b.reshape(S, 1, 128))
    return _bn(zp, st, g.reshape(S, 1, 128), be.reshape(S, 1, 128))


def kernel(x, edge_index, W1, a_src1, a_dst1, b1, g1, be1,
           W2, a_src2, a_dst2, b2, g2, be2,
           W3, a_src3, a_dst3, b3, g3, be3):
    E = edge_index.shape[1]
    EP = ((E + NW * LPW - 1) // (NW * LPW)) * (NW * LPW)
    BPT = EP // (NW * LPW)
    ei = edge_index.astype(jnp.int32)
    src = jnp.concatenate([ei[0], jnp.zeros((EP - E,), jnp.int32)])
    dst = jnp.concatenate([ei[1], jnp.full((EP - E,), MP - 1, jnp.int32)])
    src3 = src.reshape(NW, BPT, LPW)
    dst3 = dst.reshape(NW, BPT, LPW)

    z = jnp.pad(x, ((0, MP - N), (0, 0)))
    z = _layer(z, src3, dst3, W1, a_src1, a_dst1, b1, g1, be1, HEADS)
    z = _layer(z, src3, dst3, W2, a_src2, a_dst2, b2, g2, be2, HEADS)
    z = _layer(z, src3, dst3, W3, a_src3, a_dst3, b3, g3, be3, 1)
    return z[:N]
